# baseline (speedup 1.0000x reference)
import jax
import jax.numpy as jnp
from jax.experimental import pallas as pl
from jax.experimental.pallas import tpu as pltpu


HIDDEN = 64
HEADS = 4
HEAD_DIM = HIDDEN // HEADS
K = 32
M = 1000000
ACTION_DIM = 16
R = 1000
C = 1000
SEQ = 2 * K + 2
SEQP = 72

_NEG_BIG = float('-inf')

_STOCH_SQ_CACHE = []


def _stoch_sq():
    if not _STOCH_SQ_CACHE:
        st = jax.random.uniform(jax.random.key(42), (M,), dtype=jnp.float32)
        smin = jnp.min(st)
        smax = jnp.max(st)
        sn = (st - smin) / (smax - smin + 1e-12)
        _STOCH_SQ_CACHE.append(jax.device_get(sn * sn))
    return jnp.asarray(_STOCH_SQ_CACHE[0]).reshape(R, C)


_BR = 8
_BN = _BR * C


def _sims_body(m_ref, xn_ref, out_ref):
    m = m_ref[...]
    xn = xn_ref[...]
    xnb = xn.astype(jnp.bfloat16).astype(jnp.float32)
    ones = jnp.ones((1, HIDDEN), jnp.float32)
    s2 = jax.lax.dot_general(m * m, ones, (((1,), (1,)), ((), ())),
                             preferred_element_type=jnp.float32)
    mn = m / (jnp.sqrt(s2) + 1e-8)
    mnb = mn.astype(jnp.bfloat16).astype(jnp.float32)
    dp = jax.lax.dot_general(mnb, xnb, (((1,), (1,)), ((), ())),
                             preferred_element_type=jnp.float32)
    for rr in range(_BR):
        chunk = jax.lax.slice(dp, (rr * C, 0), ((rr + 1) * C, 1))
        out_ref[rr:rr + 1, :] = jnp.swapaxes(chunk, 0, 1)


def _compute_sims(mem, xn):
    return pl.pallas_call(
        _sims_body,
        grid=(M // _BN,),
        in_specs=[
            pl.BlockSpec((_BN, HIDDEN), lambda i: (i, 0)),
            pl.BlockSpec((1, HIDDEN), lambda i: (0, 0)),
        ],
        out_specs=pl.BlockSpec((_BR, C), lambda i: (i, 0)),
        out_shape=jax.ShapeDtypeStruct((R, C), jnp.float32),
    )(mem, xn)


def _topk_body(sims_ref, surp_ref, st2_ref, rew_ref, act_ref,
               memv_ref, memn_ref,
               idxo_ref, rewo_ref, acto_ref, sel_ref, nxt_ref,
               scores_ref, bmax_ref, idx_ref, sems):
    sims = sims_ref[...]
    surp = surp_ref[...]
    si = (sims - jnp.min(sims)) / (jnp.max(sims) - jnp.min(sims) + 1e-12)
    su = (surp - jnp.min(surp)) / (jnp.max(surp) - jnp.min(surp) + 1e-12)
    scores = su * su + si * si + st2_ref[...]
    scores_ref[...] = scores
    bmax_ref[...] = jnp.max(scores, axis=1, keepdims=True)

    rio = jax.lax.broadcasted_iota(jnp.int32, (R, 1), 0)
    cio = jax.lax.broadcasted_iota(jnp.int32, (1, C), 1)

    def body(k, _):
        bm = bmax_ref[...]
        mval = jnp.max(bm)
        b = jnp.min(jnp.where(bm == mval, rio, jnp.int32(1 << 30)))
        row = scores_ref[pl.ds(b, 1), :]
        rm = jnp.max(row)
        o = jnp.min(jnp.where(row == rm, cio, jnp.int32(1 << 30)))
        sel = cio == o
        g = b * C + o
        idx_ref[k] = g
        idxo_ref[pl.ds(k, 1), :] = g[None, None]
        rrow = rew_ref[pl.ds(b, 1), :]
        arow = act_ref[pl.ds(b, 1), :]
        rewo_ref[pl.ds(k, 1), :] = jnp.sum(
            jnp.where(sel, rrow, 0.0), axis=1, keepdims=True)
        acto_ref[pl.ds(k, 1), :] = jnp.sum(
            jnp.where(sel, arow, 0), axis=1, keepdims=True)
        nrow = jnp.where(sel, _NEG_BIG, row)
        scores_ref[pl.ds(b, 1), :] = nrow
        bmax_ref[pl.ds(b, 1), :] = jnp.max(nrow).reshape(1, 1)
        return 0

    jax.lax.fori_loop(0, K, body, 0)

    copies = []
    for k in range(K):
        g = idx_ref[k]
        for j, (src, dst) in enumerate(((memv_ref, sel_ref),
                                        (memn_ref, nxt_ref))):
            cp = pltpu.make_async_copy(src.at[pl.ds(g, 1)],
                                       dst.at[pl.ds(k, 1)],
                                       sems.at[(2 * k + j) % 8])
            cp.start()
            copies.append(cp)
    for cp in copies:
        cp.wait()


def _topk(sims, surp2, st2, rew2, act2, memv, memn):
    vfull = pl.BlockSpec(memory_space=pltpu.MemorySpace.VMEM)
    hspec = pl.BlockSpec(memory_space=pltpu.MemorySpace.HBM)
    return pl.pallas_call(
        _topk_body,
        in_specs=[vfull] * 5 + [hspec, hspec],
        out_specs=[vfull] * 5,
        out_shape=[
            jax.ShapeDtypeStruct((K, 1), jnp.int32),
            jax.ShapeDtypeStruct((K, 1), jnp.float32),
            jax.ShapeDtypeStruct((K, 1), jnp.int32),
            jax.ShapeDtypeStruct((K, HIDDEN), jnp.float32),
            jax.ShapeDtypeStruct((K, HIDDEN), jnp.float32),
        ],
        scratch_shapes=[
            pltpu.VMEM((R, C), jnp.float32),
            pltpu.VMEM((R, 1), jnp.float32),
            pltpu.SMEM((K,), jnp.int32),
            pltpu.SemaphoreType.DMA((8,)),
        ],
    )(sims, surp2, st2, rew2, act2, memv, memn)


def _gelu(v):
    return 0.5 * v * (1.0 + jax.lax.erf(v / jnp.sqrt(jnp.float32(2.0))))


def _lnorm(h, s, b):
    mu = jnp.mean(h, axis=-1, keepdims=True)
    var = jnp.mean((h - mu) * (h - mu), axis=-1, keepdims=True)
    return (h - mu) / jnp.sqrt(var + 1e-5) * s + b


def _mm(a, b):
    return jax.lax.dot_general(a, b, (((1,), (0,)), ((), ())),
                               preferred_element_type=jnp.float32)


def _mm_t(a, b):
    return jax.lax.dot_general(a, b, (((1,), (1,)), ((), ())),
                               preferred_element_type=jnp.float32)


_N_LAYER_REFS = 12


def _transformer_body(sel_ref, act_ref, rew_ref, x_ref, st_ref, pos_ref,
                      aW1_ref, ab1_ref, aW2_ref, ab2_ref,
                      vW1_ref, vb1_ref, vW2_ref, vb2_ref,
                      *rest):
    lrefs = rest[:2 * _N_LAYER_REFS]
    seq_out_ref, vp_ref, loss_ref, seq_ref = rest[2 * _N_LAYER_REFS:]

    oh = (jax.lax.broadcasted_iota(jnp.int32, (K, ACTION_DIM), 1)
          == act_ref[...]).astype(jnp.float32)
    act = _mm(_gelu(_mm(oh, aW1_ref[...]) + ab1_ref[...]),
              aW2_ref[...]) + ab2_ref[...]
    sel = sel_ref[...]

    seq_ref[...] = jnp.zeros((SEQP, HIDDEN), jnp.float32)
    seq_ref[0:1, :] = st_ref[...]
    for j in range(K):
        seq_ref[1 + 2 * j:2 + 2 * j, :] = sel[j:j + 1, :]
        seq_ref[2 + 2 * j:3 + 2 * j, :] = act[j:j + 1, :]
    seq_ref[SEQ - 1:SEQ, :] = x_ref[...]

    h = seq_ref[...] + pos_ref[...]

    rio = jax.lax.broadcasted_iota(jnp.int32, (SEQP, SEQP), 0)
    cio = jax.lax.broadcasted_iota(jnp.int32, (SEQP, SEQP), 1)
    mask = jnp.where(cio <= rio, 0.0, _NEG_BIG).astype(jnp.float32)

    for li in range(2):
        (ln1s, ln1b, Wqkv, bqkv, Wo, bo,
         ln2s, ln2b, Wf1, bf1, Wf2, bf2) = \
            lrefs[li * _N_LAYER_REFS:(li + 1) * _N_LAYER_REFS]
        hn = _lnorm(h, ln1s[...], ln1b[...])
        qkv = _mm(hn, Wqkv[...]) + bqkv[...]
        q = qkv[:, 0:HIDDEN]
        kk = qkv[:, HIDDEN:2 * HIDDEN]
        v = qkv[:, 2 * HIDDEN:3 * HIDDEN]
        outs = []
        for hh in range(HEADS):
            sl = slice(hh * HEAD_DIM, (hh + 1) * HEAD_DIM)
            qh, kh, vh = q[:, sl], kk[:, sl], v[:, sl]
            att = _mm_t(qh, kh) / jnp.sqrt(jnp.float32(HEAD_DIM)) + mask
            att = att - jnp.max(att, axis=-1, keepdims=True)
            e = jnp.exp(att)
            att = e / jnp.sum(e, axis=-1, keepdims=True)
            outs.append(_mm(att, vh))
        o = jnp.concatenate(outs, axis=1)
        h = h + _mm(o, Wo[...]) + bo[...]
        z = _lnorm(h, ln2s[...], ln2b[...])
        z = _mm(_gelu(_mm(z, Wf1[...]) + bf1[...]), Wf2[...]) + bf2[...]
        h = h + z

    seq_out_ref[...] = h

    vp = _mm(_gelu(_mm(h, vW1_ref[...]) + vb1_ref[...]),
             vW2_ref[...]) + vb2_ref[...]
    vp_ref[...] = vp[SEQ - 1:SEQ, :]

    prio = jax.lax.broadcasted_iota(jnp.int32, (K, SEQP), 0)
    pcio = jax.lax.broadcasted_iota(jnp.int32, (K, SEQP), 1)
    P = (pcio == 2 * prio + 1).astype(jnp.float32)
    vp_sel = _mm(P, vp)
    d = vp_sel - rew_ref[...]
    loss_ref[...] = d * d


def _transformer(sel, act, rew, x, params):
    p = params
    ins = [sel, act, rew, x,
           p['start_token'],
           jnp.pad(p['pos_emb'], ((0, SEQP - SEQ), (0, 0))),
           p['act_W1'], p['act_b1'].reshape(1, -1),
           p['act_W2'], p['act_b2'].reshape(1, -1),
           p['vh_W1'], p['vh_b1'].reshape(1, -1),
           p['vh_W2'], p['vh_b2'].reshape(1, -1)]
    for lp in p['layers']:
        ins += [lp['ln1_s'].reshape(1, -1), lp['ln1_b'].reshape(1, -1),
                lp['Wqkv'], lp['bqkv'].reshape(1, -1),
                lp['Wo'], lp['bo'].reshape(1, -1),
                lp['ln2_s'].reshape(1, -1), lp['ln2_b'].reshape(1, -1),
                lp['Wf1'], lp['bf1'].reshape(1, -1),
                lp['Wf2'], lp['bf2'].reshape(1, -1)]
    vfull = pl.BlockSpec(memory_space=pltpu.MemorySpace.VMEM)
    return pl.pallas_call(
        _transformer_body,
        in_specs=[vfull] * len(ins),
        out_specs=[vfull, vfull, vfull],
        out_shape=[
            jax.ShapeDtypeStruct((SEQP, HIDDEN), jnp.float32),
            jax.ShapeDtypeStruct((1, 1), jnp.float32),
            jax.ShapeDtypeStruct((K, 1), jnp.float32),
        ],
        scratch_shapes=[pltpu.VMEM((SEQP, HIDDEN), jnp.float32)],
    )(*ins)


def kernel(x, memory_vectors, memory_next_vectors, memory_surprises,
           memory_rewards, memory_actions, params):
    xn = x / (jnp.linalg.norm(x, axis=-1, keepdims=True) + 1e-8)
    sims = _compute_sims(memory_vectors, xn)

    idx, rew, act, sel, nxt = _topk(
        sims,
        memory_surprises.reshape(R, C),
        _stoch_sq(),
        memory_rewards.reshape(R, C),
        memory_actions.reshape(R, C),
        memory_vectors,
        memory_next_vectors,
    )
    del idx

    seq, vp_last, losses = _transformer(sel, act, rew, x, params)

    return (seq[1:SEQ], nxt, vp_last.reshape(1),
            losses.reshape(K), rew.reshape(K))


# speedup vs baseline: 1.0052x; 1.0052x over previous
import jax
import jax.numpy as jnp
from jax.experimental import pallas as pl
from jax.experimental.pallas import tpu as pltpu


HIDDEN = 64
HEADS = 4
HEAD_DIM = HIDDEN // HEADS
K = 32
M = 1000000
ACTION_DIM = 16
R = 1000
C = 1000
SEQ = 2 * K + 2
SEQP = 72
_RP = 1024

_NEG_BIG = float('-inf')

_STOCH_SQ_CACHE = []


def _stoch_sq():
    if not _STOCH_SQ_CACHE:
        st = jax.random.uniform(jax.random.key(42), (M,), dtype=jnp.float32)
        smin = jnp.min(st)
        smax = jnp.max(st)
        sn = (st - smin) / (smax - smin + 1e-12)
        _STOCH_SQ_CACHE.append(jax.device_get(sn * sn))
    return jnp.asarray(_STOCH_SQ_CACHE[0]).reshape(R, C)


_BR = 8
_BN = _BR * C


def _sims_body(m_ref, xn_ref, out_ref):
    m = m_ref[...]
    xn = xn_ref[...]
    xnb = xn.astype(jnp.bfloat16).astype(jnp.float32)
    ones = jnp.ones((1, HIDDEN), jnp.float32)
    s2 = jax.lax.dot_general(m * m, ones, (((1,), (1,)), ((), ())),
                             preferred_element_type=jnp.float32)
    mn = m / (jnp.sqrt(s2) + 1e-8)
    mnb = mn.astype(jnp.bfloat16).astype(jnp.float32)
    dp = jax.lax.dot_general(mnb, xnb, (((1,), (1,)), ((), ())),
                             preferred_element_type=jnp.float32)
    for rr in range(_BR):
        chunk = jax.lax.slice(dp, (rr * C, 0), ((rr + 1) * C, 1))
        out_ref[rr:rr + 1, :] = jnp.swapaxes(chunk, 0, 1)


def _compute_sims(mem, xn):
    return pl.pallas_call(
        _sims_body,
        grid=(M // _BN,),
        in_specs=[
            pl.BlockSpec((_BN, HIDDEN), lambda i: (i, 0)),
            pl.BlockSpec((1, HIDDEN), lambda i: (0, 0)),
        ],
        out_specs=pl.BlockSpec((_BR, C), lambda i: (i, 0)),
        out_shape=jax.ShapeDtypeStruct((R, C), jnp.float32),
    )(mem, xn)


def _topk_body(sims_ref, surp_ref, st2_ref, rew_ref, act_ref,
               memv_ref, memn_ref,
               idxo_ref, rewo_ref, acto_ref, sel_ref, nxt_ref,
               scores_ref, bmax_ref, idx_ref, sems):
    sims = sims_ref[...]
    surp = surp_ref[...]
    si = (sims - jnp.min(sims)) / (jnp.max(sims) - jnp.min(sims) + 1e-12)
    su = (surp - jnp.min(surp)) / (jnp.max(surp) - jnp.min(surp) + 1e-12)
    scores = su * su + si * si + st2_ref[...]
    scores_ref[...] = scores
    rmax = jnp.max(scores, axis=1, keepdims=True)
    bmax_ref[...] = jnp.concatenate(
        [jnp.swapaxes(rmax, 0, 1),
         jnp.full((1, _RP - R), _NEG_BIG, jnp.float32)], axis=1)

    bio = jax.lax.broadcasted_iota(jnp.int32, (1, _RP), 1)
    cio = jax.lax.broadcasted_iota(jnp.int32, (1, C), 1)

    def body(k, _):
        bm = bmax_ref[...]
        bsel = bm == jnp.max(bm, axis=1, keepdims=True)
        b = jnp.min(jnp.where(bsel, bio, jnp.int32(1 << 30)))
        row = scores_ref[pl.ds(b, 1), :]
        rm = jnp.max(row, axis=1, keepdims=True)
        o = jnp.min(jnp.where(row == rm, cio, jnp.int32(1 << 30)))
        sel = cio == o[None, None]
        g = b * C + o
        idx_ref[k] = g
        idxo_ref[pl.ds(k, 1), :] = g[None, None]
        rrow = rew_ref[pl.ds(b, 1), :]
        arow = act_ref[pl.ds(b, 1), :]
        rewo_ref[pl.ds(k, 1), :] = jnp.sum(
            jnp.where(sel, rrow, 0.0), axis=1, keepdims=True)
        acto_ref[pl.ds(k, 1), :] = jnp.sum(
            jnp.where(sel, arow, 0), axis=1, keepdims=True)
        nrow = jnp.where(sel, _NEG_BIG, row)
        scores_ref[pl.ds(b, 1), :] = nrow
        bmax_ref[...] = jnp.where(bio == b, jnp.max(nrow), bm)
        return 0

    jax.lax.fori_loop(0, K, body, 0)

    copies = []
    for k in range(K):
        g = idx_ref[k]
        for j, (src, dst) in enumerate(((memv_ref, sel_ref),
                                        (memn_ref, nxt_ref))):
            cp = pltpu.make_async_copy(src.at[pl.ds(g, 1)],
                                       dst.at[pl.ds(k, 1)],
                                       sems.at[(2 * k + j) % 8])
            cp.start()
            copies.append(cp)
    for cp in copies:
        cp.wait()


def _topk(sims, surp2, st2, rew2, act2, memv, memn):
    vfull = pl.BlockSpec(memory_space=pltpu.MemorySpace.VMEM)
    hspec = pl.BlockSpec(memory_space=pltpu.MemorySpace.HBM)
    return pl.pallas_call(
        _topk_body,
        in_specs=[vfull] * 5 + [hspec, hspec],
        out_specs=[vfull] * 5,
        out_shape=[
            jax.ShapeDtypeStruct((K, 1), jnp.int32),
            jax.ShapeDtypeStruct((K, 1), jnp.float32),
            jax.ShapeDtypeStruct((K, 1), jnp.int32),
            jax.ShapeDtypeStruct((K, HIDDEN), jnp.float32),
            jax.ShapeDtypeStruct((K, HIDDEN), jnp.float32),
        ],
        scratch_shapes=[
            pltpu.VMEM((R, C), jnp.float32),
            pltpu.VMEM((1, _RP), jnp.float32),
            pltpu.SMEM((K,), jnp.int32),
            pltpu.SemaphoreType.DMA((8,)),
        ],
    )(sims, surp2, st2, rew2, act2, memv, memn)


def _gelu(v):
    return 0.5 * v * (1.0 + jax.lax.erf(v / jnp.sqrt(jnp.float32(2.0))))


def _lnorm(h, s, b):
    mu = jnp.mean(h, axis=-1, keepdims=True)
    var = jnp.mean((h - mu) * (h - mu), axis=-1, keepdims=True)
    return (h - mu) / jnp.sqrt(var + 1e-5) * s + b


def _mm(a, b):
    return jax.lax.dot_general(a, b, (((1,), (0,)), ((), ())),
                               preferred_element_type=jnp.float32)


def _mm_t(a, b):
    return jax.lax.dot_general(a, b, (((1,), (1,)), ((), ())),
                               preferred_element_type=jnp.float32)


_N_LAYER_REFS = 12


def _transformer_body(sel_ref, act_ref, rew_ref, x_ref, st_ref, pos_ref,
                      aW1_ref, ab1_ref, aW2_ref, ab2_ref,
                      vW1_ref, vb1_ref, vW2_ref, vb2_ref,
                      *rest):
    lrefs = rest[:2 * _N_LAYER_REFS]
    seq_out_ref, vp_ref, loss_ref, seq_ref = rest[2 * _N_LAYER_REFS:]

    oh = (jax.lax.broadcasted_iota(jnp.int32, (K, ACTION_DIM), 1)
          == act_ref[...]).astype(jnp.float32)
    act = _mm(_gelu(_mm(oh, aW1_ref[...]) + ab1_ref[...]),
              aW2_ref[...]) + ab2_ref[...]
    sel = sel_ref[...]

    seq_ref[...] = jnp.zeros((SEQP, HIDDEN), jnp.float32)
    seq_ref[0:1, :] = st_ref[...]
    for j in range(K):
        seq_ref[1 + 2 * j:2 + 2 * j, :] = sel[j:j + 1, :]
        seq_ref[2 + 2 * j:3 + 2 * j, :] = act[j:j + 1, :]
    seq_ref[SEQ - 1:SEQ, :] = x_ref[...]

    h = seq_ref[...] + pos_ref[...]

    rio = jax.lax.broadcasted_iota(jnp.int32, (SEQP, SEQP), 0)
    cio = jax.lax.broadcasted_iota(jnp.int32, (SEQP, SEQP), 1)
    mask = jnp.where(cio <= rio, 0.0, _NEG_BIG).astype(jnp.float32)

    for li in range(2):
        (ln1s, ln1b, Wqkv, bqkv, Wo, bo,
         ln2s, ln2b, Wf1, bf1, Wf2, bf2) = \
            lrefs[li * _N_LAYER_REFS:(li + 1) * _N_LAYER_REFS]
        hn = _lnorm(h, ln1s[...], ln1b[...])
        qkv = _mm(hn, Wqkv[...]) + bqkv[...]
        q = qkv[:, 0:HIDDEN]
        kk = qkv[:, HIDDEN:2 * HIDDEN]
        v = qkv[:, 2 * HIDDEN:3 * HIDDEN]
        outs = []
        for hh in range(HEADS):
            sl = slice(hh * HEAD_DIM, (hh + 1) * HEAD_DIM)
            qh, kh, vh = q[:, sl], kk[:, sl], v[:, sl]
            att = _mm_t(qh, kh) / jnp.sqrt(jnp.float32(HEAD_DIM)) + mask
            att = att - jnp.max(att, axis=-1, keepdims=True)
            e = jnp.exp(att)
            att = e / jnp.sum(e, axis=-1, keepdims=True)
            outs.append(_mm(att, vh))
        o = jnp.concatenate(outs, axis=1)
        h = h + _mm(o, Wo[...]) + bo[...]
        z = _lnorm(h, ln2s[...], ln2b[...])
        z = _mm(_gelu(_mm(z, Wf1[...]) + bf1[...]), Wf2[...]) + bf2[...]
        h = h + z

    seq_out_ref[...] = h

    vp = _mm(_gelu(_mm(h, vW1_ref[...]) + vb1_ref[...]),
             vW2_ref[...]) + vb2_ref[...]
    vp_ref[...] = vp[SEQ - 1:SEQ, :]

    prio = jax.lax.broadcasted_iota(jnp.int32, (K, SEQP), 0)
    pcio = jax.lax.broadcasted_iota(jnp.int32, (K, SEQP), 1)
    P = (pcio == 2 * prio + 1).astype(jnp.float32)
    vp_sel = _mm(P, vp)
    d = vp_sel - rew_ref[...]
    loss_ref[...] = d * d


def _transformer(sel, act, rew, x, params):
    p = params
    ins = [sel, act, rew, x,
           p['start_token'],
           jnp.pad(p['pos_emb'], ((0, SEQP - SEQ), (0, 0))),
           p['act_W1'], p['act_b1'].reshape(1, -1),
           p['act_W2'], p['act_b2'].reshape(1, -1),
           p['vh_W1'], p['vh_b1'].reshape(1, -1),
           p['vh_W2'], p['vh_b2'].reshape(1, -1)]
    for lp in p['layers']:
        ins += [lp['ln1_s'].reshape(1, -1), lp['ln1_b'].reshape(1, -1),
                lp['Wqkv'], lp['bqkv'].reshape(1, -1),
                lp['Wo'], lp['bo'].reshape(1, -1),
                lp['ln2_s'].reshape(1, -1), lp['ln2_b'].reshape(1, -1),
                lp['Wf1'], lp['bf1'].reshape(1, -1),
                lp['Wf2'], lp['bf2'].reshape(1, -1)]
    vfull = pl.BlockSpec(memory_space=pltpu.MemorySpace.VMEM)
    return pl.pallas_call(
        _transformer_body,
        in_specs=[vfull] * len(ins),
        out_specs=[vfull, vfull, vfull],
        out_shape=[
            jax.ShapeDtypeStruct((SEQP, HIDDEN), jnp.float32),
            jax.ShapeDtypeStruct((1, 1), jnp.float32),
            jax.ShapeDtypeStruct((K, 1), jnp.float32),
        ],
        scratch_shapes=[pltpu.VMEM((SEQP, HIDDEN), jnp.float32)],
    )(*ins)


def kernel(x, memory_vectors, memory_next_vectors, memory_surprises,
           memory_rewards, memory_actions, params):
    xn = x / (jnp.linalg.norm(x, axis=-1, keepdims=True) + 1e-8)
    sims = _compute_sims(memory_vectors, xn)

    idx, rew, act, sel, nxt = _topk(
        sims,
        memory_surprises.reshape(R, C),
        _stoch_sq(),
        memory_rewards.reshape(R, C),
        memory_actions.reshape(R, C),
        memory_vectors,
        memory_next_vectors,
    )
    del idx

    seq, vp_last, losses = _transformer(sel, act, rew, x, params)

    return (seq[1:SEQ], nxt, vp_last.reshape(1),
            losses.reshape(K), rew.reshape(K))
